# baseline (speedup 1.0000x reference)
import functools

import jax
import jax.numpy as jnp
from jax import lax
from jax.experimental import pallas as pl
from jax.experimental.pallas import tpu as pltpu
from jax.experimental.pallas import tpu_sc as plsc

B = 16384
C = 1000
FILL = 1.0 / (C - 1)

NC = 2
NS = 16
NW = NC * NS
CPW = B // NW
CB = 128
NJ = CPW // CB
HA = 496
HB = C - HA
L = 16


@functools.partial(
    pl.kernel,
    mesh=plsc.VectorSubcoreMesh(core_axis_name="c", subcore_axis_name="s"),
    out_type=jax.ShapeDtypeStruct((C, B), jnp.float32),
    scratch_types=[
        pltpu.VMEM((HA, CB), jnp.float32),
        pltpu.VMEM((HB, CB), jnp.float32),
        pltpu.VMEM((CPW,), jnp.int32),
        pltpu.SemaphoreType.DMA,
        pltpu.SemaphoreType.DMA,
    ],
    compiler_params=pltpu.CompilerParams(
        skip_device_barrier=True,
        disable_bounds_checks=True,
        disable_semaphore_checks=True,
    ),
)
def _virtual_teacher(y_hbm, out_hbm, buf_a, buf_b, yv, sem_a, sem_b):
    wid = lax.axis_index("s") * NC + lax.axis_index("c")
    base = wid * CPW

    pltpu.sync_copy(y_hbm.at[pl.ds(base, CPW)], yv)

    fill_vec = jnp.full((L,), FILL, dtype=jnp.float32)
    iota = lax.iota(jnp.int32, L)

    def fill_a(r, carry):
        for k in range(CB // L):
            buf_a[r, pl.ds(k * L, L)] = fill_vec
        return carry

    def fill_b(r, carry):
        for k in range(CB // L):
            buf_b[r, pl.ds(k * L, L)] = fill_vec
        return carry

    def scan_pass(buf, h0, hrows, j, value):
        def group(g, carry):
            ys = yv[pl.ds(j * CB + g * L, L)]
            cstart = pl.multiple_of(g * L, L)
            for jj in range(L):
                y_r = ys[jj]
                hit = jnp.logical_and(y_r >= h0, y_r < h0 + hrows)
                row = jnp.clip(y_r - h0, 0, hrows - 1)
                lane = jnp.where(hit, jj, -1)
                old = buf[row, pl.ds(cstart, L)]
                buf[row, pl.ds(cstart, L)] = jnp.where(iota == lane, value, old)
            return carry

        lax.fori_loop(0, CB // L, group, 0)

    def fire(buf, h0, j, sem):
        return pltpu.async_copy(
            buf,
            out_hbm.at[pl.ds(h0, buf.shape[0]), pl.ds(base + j * CB, CB)],
            sem,
        )

    def drain(buf, h0, sem):
        pltpu.make_async_copy(
            buf, out_hbm.at[pl.ds(h0, buf.shape[0]), pl.ds(base, CB)], sem
        ).wait()

    lax.fori_loop(0, HA, fill_a, 0)
    scan_pass(buf_a, 0, HA, 0, 0.0)
    fire(buf_a, 0, 0, sem_a)
    lax.fori_loop(0, HB, fill_b, 0)
    scan_pass(buf_b, HA, HB, 0, 0.0)
    fire(buf_b, HA, 0, sem_b)

    def chunk_body(j, carry):
        drain(buf_a, 0, sem_a)
        scan_pass(buf_a, 0, HA, j - 1, FILL)
        scan_pass(buf_a, 0, HA, j, 0.0)
        fire(buf_a, 0, j, sem_a)
        drain(buf_b, HA, sem_b)
        scan_pass(buf_b, HA, HB, j - 1, FILL)
        scan_pass(buf_b, HA, HB, j, 0.0)
        fire(buf_b, HA, j, sem_b)
        return carry

    lax.fori_loop(1, NJ, chunk_body, 0)

    drain(buf_a, 0, sem_a)
    drain(buf_b, HA, sem_b)


def kernel(x, y):
    del x
    return _virtual_teacher(y.astype(jnp.int32)).T
